# baseline (speedup 1.0000x reference)
import functools

import jax
import jax.numpy as jnp
from jax import lax
from jax.experimental import pallas as pl
from jax.experimental.pallas import tpu as pltpu
from jax.experimental.pallas import tpu_sc as plsc

NUM_EMB = 1000000
DIM = 64
BATCH = 16384
NUM_WORKERS = 32
CW = 256
NCH0 = 122
SPAN = NCH0 * CW
TAIL_LO = NUM_EMB - DIM
TAIL_START = NUM_EMB - 128
CAP = 16384
SCAP = 18304
OUT_ROWS = BATCH + 256
BATCH_ROWS = 112


def kernel(x, weight):
    wt = weight.T
    tail = lax.slice(wt, (0, TAIL_START), (DIM, NUM_EMB))
    mesh = plsc.VectorSubcoreMesh(core_axis_name="c", subcore_axis_name="s")

    @functools.partial(
        pl.kernel,
        mesh=mesh,
        out_type=jax.ShapeDtypeStruct((OUT_ROWS, 128), jnp.float32),
        scratch_types=[
            pltpu.VMEM((2048,), jnp.int32),
            pltpu.VMEM((2048,), jnp.int32),
            pltpu.VMEM((CAP,), jnp.int32),
            pltpu.VMEM((SCAP,), jnp.int32),
            pltpu.VMEM((128,), jnp.int32),
            pltpu.VMEM((DIM, CW), jnp.float32),
            pltpu.VMEM((DIM, CW), jnp.float32),
            pltpu.VMEM((DIM, CW), jnp.float32),
            pltpu.VMEM((DIM, 128), jnp.float32),
            pltpu.VMEM((BATCH_ROWS, 128), jnp.float32),
            pltpu.VMEM((BATCH_ROWS,), jnp.int32),
            pltpu.SMEM((128,), jnp.int32),
            pltpu.SMEM((128,), jnp.int32),
            pltpu.SMEM((128,), jnp.int32),
            pltpu.SMEM((8,), jnp.int32),
            *[pltpu.SemaphoreType.DMA for _ in range(7)],
        ],
        compiler_params=pltpu.CompilerParams(needs_layout_passes=False),
    )
    def body(x_hbm, w_hbm, tail_hbm, out_hbm, xb0, xb1, pk_v,
             sp_v, hist_v, cb0, cb1, cb2, tail_v, rows_v, blist_v,
             base_s, fill_s, cnt_s, m_s, sem0, sem1, sem2, fsem, xsem0,
             xsem1, tsem):
        wid = lax.axis_index("s") * 2 + lax.axis_index("c")
        lo = wid * SPAN
        is_last = wid == NUM_WORKERS - 1
        nch = NCH0 + jnp.where(is_last, 2, 0).astype(jnp.int32)
        hi = lo + nch * CW + jnp.where(is_last, DIM, 0).astype(jnp.int32)

        cbs = (cb0, cb1, cb2)
        sems = (sem0, sem1, sem2)
        iota = lax.iota(jnp.int32, 16)
        lane0 = iota == 0
        ones = jnp.full((16,), 1, jnp.int32)
        dvecs = [iota + 16 * q for q in range(4)]

        tail_cp = pltpu.make_async_copy(tail_hbm, tail_v, tsem)
        tail_cp.start()

        def fire(cidx, bb):
            start = pl.multiple_of(
                lo + jnp.minimum(cidx, nch - 1) * CW, 128)
            pltpu.async_copy(
                w_hbm.at[:, pl.ds(start, CW)], cbs[bb], sems[bb]
            )

        fire(jnp.int32(0), 0)
        fire(jnp.int32(1), 1)

        m_s[0] = jnp.int32(0)

        def refill_dumps():
            for j in range(BATCH_ROWS // 16):
                blist_v[pl.ds(j * 16, 16)] = BATCH + 16 * j + iota

        refill_dumps()

        losp = jnp.full((16,), lo, jnp.int32)
        hisp = jnp.full((16,), hi, jnp.int32)

        carry = jnp.int32(0)
        xbs = (xb0, xb1)
        xsems = (xsem0, xsem1)

        def xfire(piece, xb):
            pltpu.async_copy(
                x_hbm.at[pl.ds(piece * 2048, 2048)], xbs[xb], xsems[xb])

        xfire(0, 0)
        xfire(1, 1)
        for piece in range(8):
            xb = piece % 2
            pltpu.make_async_copy(
                x_hbm.at[pl.ds(0, 2048)], xbs[xb], xsems[xb]).wait()
            xb_v = xbs[xb]

            @pl.loop(0, 128, init_carry=carry)
            def match_loop(u, ptr):
                xv = xb_v[pl.ds(u * 16, 16)]
                bvec = piece * 2048 + u * 16 + iota
                m = jnp.logical_and(xv >= losp, xv < hisp)
                cs = plsc.cumsum(jnp.where(m, 1, 0).astype(jnp.int32))
                pos = cs + (ptr - 1)
                packed = jnp.bitwise_or(
                    lax.shift_left(xv - losp, 14), bvec)
                plsc.store_scatter(pk_v, [pos], packed, mask=m)
                return ptr + cs[15]

            if piece + 2 < 8:
                xfire(piece + 2, xb)
            carry = match_loop
        nmatch = carry

        for j in range(8):
            hist_v[pl.ds(j * 16, 16)] = jnp.zeros((16,), jnp.int32)

        nmsp = jnp.full((16,), nmatch, jnp.int32)

        ngm = lax.shift_right_logical(nmatch + 15, 4)

        @pl.loop(0, ngm)
        def hist_loop(u):
            pv = pk_v[pl.ds(u * 16, 16)]
            valid = (u * 16 + iota) < nmsp
            cvec = lax.shift_right_logical(pv, 22)
            plsc.addupdate_scatter(hist_v, [cvec], ones, mask=valid)

        acc = jnp.int32(0)
        for j in range(8):
            h = hist_v[pl.ds(j * 16, 16)]
            ha = lax.shift_left(lax.shift_right_logical(h + 15, 4), 4)
            csa = plsc.cumsum(ha)
            starts = csa - ha + acc
            for lane in range(16):
                base_s[j * 16 + lane] = starts[lane]
                fill_s[j * 16 + lane] = starts[lane]
                cnt_s[j * 16 + lane] = h[lane]
            acc = acc + csa[15]

        @pl.loop(0, ngm)
        def place_loop(u):
            pv = pk_v[pl.ds(u * 16, 16)]
            for lane in range(16):
                jj = u * 16 + lane

                @pl.when(jj < nmatch)
                def _():
                    pk = pv[lane]
                    cc = lax.shift_right_logical(pk, 22)
                    pos = fill_s[cc]
                    fill_s[cc] = pos + 1
                    plsc.store_scatter(
                        sp_v, [jnp.full((16,), pos, jnp.int32)],
                        jnp.full((16,), pk, jnp.int32), mask=lane0)

        def flush():
            pltpu.async_copy(rows_v, out_hbm.at[blist_v], fsem).wait()
            refill_dumps()
            m_s[0] = jnp.int32(0)

        def emit(b, src_ref, col):
            mrow = m_s[0]
            csp = jnp.full((16,), col, jnp.int32)
            for q in range(4):
                v = plsc.load_gather(src_ref, [dvecs[q], csp])
                rows_v[mrow, pl.ds(q * 16, 16)] = v
            plsc.store_scatter(
                blist_v, [jnp.full((16,), mrow, jnp.int32)],
                jnp.full((16,), b, jnp.int32), mask=lane0)
            m_s[0] = mrow + 1

            @pl.when(mrow + 1 == BATCH_ROWS)
            def _():
                flush()

        NCHU = NCH0 + 2

        @pl.loop(0, (NCHU + 2) // 3)
        def sweep_loop(g):
            for bb in range(3):
                cidx = g * 3 + bb

                @pl.when(cidx < NCHU)
                def _():
                    pltpu.make_async_copy(
                        w_hbm.at[:, pl.ds(0, CW)], cbs[bb], sems[bb]
                    ).wait()
                    fire(cidx + 2, (bb + 2) % 3)
                    chunk_lo = lo + cidx * CW
                    cbase = base_s[cidx]
                    cn = cnt_s[cidx]
                    ng = lax.shift_right_logical(cn + 15, 4)

                    @pl.loop(0, ng)
                    def grp(u):
                        pv = sp_v[pl.ds(cbase + u * 16, 16)]
                        for lane in range(16):
                            jj = u * 16 + lane

                            @pl.when(jj < cn)
                            def _():
                                pk = pv[lane]
                                emit(jnp.bitwise_and(pk, 16383), cbs[bb],
                                     lax.shift_right_logical(pk, 14)
                                     - cidx * CW)

        for bb in (1, 2):
            pltpu.make_async_copy(
                w_hbm.at[:, pl.ds(0, CW)], cbs[bb], sems[bb]
            ).wait()

        tail_cp.wait()
        tail_off = TAIL_START - lo
        tbase = base_s[NCH0 + 2]
        tn = cnt_s[NCH0 + 2]
        ngt = lax.shift_right_logical(tn + 15, 4)

        @pl.loop(0, ngt)
        def tail_grp(u):
            pv = sp_v[pl.ds(tbase + u * 16, 16)]
            for lane in range(16):
                jj = u * 16 + lane

                @pl.when(jj < tn)
                def _():
                    pk = pv[lane]
                    emit(jnp.bitwise_and(pk, 16383), tail_v,
                         lax.shift_right_logical(pk, 14) - tail_off)

        @pl.when(m_s[0] > 0)
        def _():
            flush()

    out = body(x.astype(jnp.int32), wt, tail)
    return lax.slice(out, (0, 0), (BATCH, DIM))


# speedup vs baseline: 1.3201x; 1.3201x over previous
import functools

import jax
import jax.numpy as jnp
from jax import lax
from jax.experimental import pallas as pl
from jax.experimental.pallas import tpu as pltpu
from jax.experimental.pallas import tpu_sc as plsc

NUM_EMB = 1000000
DIM = 64
BATCH = 16384
NUM_WORKERS = 32
CW = 512
NCH0 = 61
SPAN = NCH0 * CW
MAX_T = NUM_EMB // 128 - 1
TAIL_LO = NUM_EMB - DIM
TAIL_START = NUM_EMB - 128
CAP = 2048
SCAP = 2560
OUT_ROWS = BATCH + 256
BATCH_ROWS = 112


def kernel(x, weight):
    wt = weight.T
    tail = lax.slice(wt, (0, TAIL_START), (DIM, NUM_EMB))
    mesh = plsc.VectorSubcoreMesh(core_axis_name="c", subcore_axis_name="s")

    @functools.partial(
        pl.kernel,
        mesh=mesh,
        out_type=jax.ShapeDtypeStruct((OUT_ROWS, 128), jnp.float32),
        scratch_types=[
            pltpu.VMEM((2048,), jnp.int32),
            pltpu.VMEM((2048,), jnp.int32),
            pltpu.VMEM((CAP,), jnp.int32),
            pltpu.VMEM((SCAP,), jnp.int32),
            pltpu.VMEM((64,), jnp.int32),
            pltpu.VMEM((DIM, CW), jnp.float32),
            pltpu.VMEM((DIM, CW), jnp.float32),
            pltpu.VMEM((DIM, CW), jnp.float32),
            pltpu.VMEM((DIM, 128), jnp.float32),
            pltpu.VMEM((BATCH_ROWS, 128), jnp.float32),
            pltpu.VMEM((BATCH_ROWS,), jnp.int32),
            pltpu.SMEM((64,), jnp.int32),
            pltpu.SMEM((64,), jnp.int32),
            pltpu.SMEM((64,), jnp.int32),
            pltpu.SMEM((8,), jnp.int32),
            *[pltpu.SemaphoreType.DMA for _ in range(7)],
        ],
        compiler_params=pltpu.CompilerParams(needs_layout_passes=False),
    )
    def body(x_hbm, w_hbm, tail_hbm, out_hbm, xb0, xb1, pk_v, sp_v,
             hist_v, cb0, cb1, cb2, tail_v, rows_v, blist_v,
             base_s, fill_s, cnt_s, m_s, sem0, sem1, sem2, fsem, xsem0,
             xsem1, tsem):
        wid = lax.axis_index("s") * 2 + lax.axis_index("c")
        lo = wid * SPAN
        is_last = wid == NUM_WORKERS - 1
        nch = NCH0 + jnp.where(is_last, 1, 0).astype(jnp.int32)
        hi = lo + nch * CW + jnp.where(is_last, DIM, 0).astype(jnp.int32)

        cbs = (cb0, cb1, cb2)
        sems = (sem0, sem1, sem2)
        iota = lax.iota(jnp.int32, 16)
        lane0 = iota == 0
        ones = jnp.full((16,), 1, jnp.int32)
        dvecs = [iota + 16 * q for q in range(4)]

        tail_cp = pltpu.make_async_copy(tail_hbm, tail_v, tsem)
        tail_cp.start()

        def fire(cidx, bb):
            start = pl.multiple_of(
                lo + jnp.minimum(cidx, nch - 1) * CW, 128)
            pltpu.async_copy(
                w_hbm.at[:, pl.ds(start, CW)], cbs[bb], sems[bb]
            )

        fire(jnp.int32(0), 0)
        fire(jnp.int32(1), 1)

        m_s[0] = jnp.int32(0)

        def refill_dumps():
            for j in range(BATCH_ROWS // 16):
                blist_v[pl.ds(j * 16, 16)] = BATCH + 16 * j + iota

        refill_dumps()

        losp = jnp.full((16,), lo, jnp.int32)
        hisp = jnp.full((16,), hi, jnp.int32)
        capsp = jnp.full((16,), CAP, jnp.int32)

        carry = jnp.int32(0)
        xbs = (xb0, xb1)
        xsems = (xsem0, xsem1)

        def xfire(piece, xb):
            pltpu.async_copy(
                x_hbm.at[pl.ds(piece * 2048, 2048)], xbs[xb], xsems[xb])

        xfire(0, 0)
        xfire(1, 1)
        for piece in range(8):
            xb = piece % 2
            pltpu.make_async_copy(
                x_hbm.at[pl.ds(0, 2048)], xbs[xb], xsems[xb]).wait()
            xb_v = xbs[xb]

            @pl.loop(0, 128, init_carry=carry)
            def match_loop(u, ptr):
                xv = xb_v[pl.ds(u * 16, 16)]
                bvec = piece * 2048 + u * 16 + iota
                m = jnp.logical_and(xv >= losp, xv < hisp)
                cs = plsc.cumsum(jnp.where(m, 1, 0).astype(jnp.int32))
                pos = cs + (ptr - 1)
                packed = jnp.bitwise_or(lax.shift_left(xv - losp, 14), bvec)
                mst = jnp.logical_and(m, pos < capsp)
                plsc.store_scatter(pk_v, [pos], packed, mask=mst)
                return ptr + cs[15]

            if piece + 2 < 8:
                xfire(piece + 2, xb)
            carry = match_loop
        nraw = carry
        nmatch = jnp.minimum(nraw, jnp.int32(CAP))

        for j in range(4):
            hist_v[pl.ds(j * 16, 16)] = jnp.zeros((16,), jnp.int32)

        nmsp = jnp.full((16,), nmatch, jnp.int32)
        ngm = lax.shift_right_logical(nmatch + 15, 4)

        @pl.loop(0, ngm)
        def hist_loop(u):
            pv = pk_v[pl.ds(u * 16, 16)]
            valid = (u * 16 + iota) < nmsp
            cvec = lax.shift_right_logical(pv, 23)
            plsc.addupdate_scatter(hist_v, [cvec], ones, mask=valid)

        acc = jnp.int32(0)
        for j in range(4):
            h = hist_v[pl.ds(j * 16, 16)]
            ha = lax.shift_left(lax.shift_right_logical(h + 7, 3), 3)
            csa = plsc.cumsum(ha)
            starts = csa - ha + acc
            for lane in range(16):
                base_s[j * 16 + lane] = starts[lane]
                fill_s[j * 16 + lane] = starts[lane]
                cnt_s[j * 16 + lane] = h[lane]
            acc = acc + csa[15]

        @pl.loop(0, ngm)
        def place_loop(u):
            pv = pk_v[pl.ds(u * 16, 16)]
            for lane in range(16):
                jj = u * 16 + lane

                @pl.when(jj < nmatch)
                def _():
                    pk = pv[lane]
                    cc = lax.shift_right_logical(pk, 23)
                    pos = fill_s[cc]
                    fill_s[cc] = pos + 1
                    plsc.store_scatter(
                        sp_v, [jnp.full((16,), pos, jnp.int32)],
                        jnp.full((16,), pk, jnp.int32), mask=lane0)

        def flush():
            pltpu.async_copy(rows_v, out_hbm.at[blist_v], fsem).wait()
            refill_dumps()
            m_s[0] = jnp.int32(0)

        def emit(b, src_ref, col):
            mrow = m_s[0]
            csp = jnp.full((16,), col, jnp.int32)
            for q in range(4):
                v = plsc.load_gather(src_ref, [dvecs[q], csp])
                rows_v[mrow, pl.ds(q * 16, 16)] = v
            plsc.store_scatter(
                blist_v, [jnp.full((16,), mrow, jnp.int32)],
                jnp.full((16,), b, jnp.int32), mask=lane0)
            m_s[0] = mrow + 1

            @pl.when(mrow + 1 == BATCH_ROWS)
            def _():
                flush()

        @pl.when(nraw > CAP)
        def _():
            tail_cp.wait()
            m_s[1] = jnp.int32(0)

            @pl.loop(0, 1024)
            def slow_loop(u):
                pltpu.sync_copy(
                    x_hbm.at[pl.ds(u * 16, 16)], xb0.at[pl.ds(0, 16)])
                xv = xb0[pl.ds(0, 16)]
                for lane in range(16):
                    i = xv[lane]
                    b = u * 16 + lane

                    @pl.when(jnp.logical_and(i >= lo, i < hi))
                    def _():
                        rank = m_s[1]
                        m_s[1] = rank + 1

                        @pl.when(rank >= CAP)
                        def _():
                            @pl.when(i >= TAIL_LO)
                            def _():
                                pltpu.async_copy(
                                    tail_hbm, tail_v, tsem).wait()
                                emit(b, tail_v, i - TAIL_START)

                            @pl.when(i < TAIL_LO)
                            def _():
                                t = jnp.minimum(
                                    lax.shift_right_logical(i, 7),
                                    jnp.int32(MAX_T))
                                start = pl.multiple_of(t * 128, 128)
                                pltpu.async_copy(
                                    w_hbm.at[:, pl.ds(start, 128)],
                                    tail_v, tsem).wait()
                                emit(b, tail_v, i - start)

            pltpu.async_copy(tail_hbm, tail_v, tsem).wait()

        NCHU = NCH0 + 1

        @pl.loop(0, (NCHU + 2) // 3)
        def sweep_loop(g):
            for bb in range(3):
                cidx = g * 3 + bb

                @pl.when(cidx < NCHU)
                def _():
                    pltpu.make_async_copy(
                        w_hbm.at[:, pl.ds(0, CW)], cbs[bb], sems[bb]
                    ).wait()
                    fire(cidx + 2, (bb + 2) % 3)
                    cbase = base_s[cidx]
                    cn = cnt_s[cidx]
                    ng = lax.shift_right_logical(cn + 15, 4)

                    @pl.loop(0, ng)
                    def grp(u):
                        pv = sp_v[pl.ds(cbase + u * 16, 16)]
                        for lane in range(16):
                            jj = u * 16 + lane

                            @pl.when(jj < cn)
                            def _():
                                pk = pv[lane]
                                emit(jnp.bitwise_and(pk, 16383), cbs[bb],
                                     lax.shift_right_logical(pk, 14)
                                     - cidx * CW)

        for bb in (2, 0):
            pltpu.make_async_copy(
                w_hbm.at[:, pl.ds(0, CW)], cbs[bb], sems[bb]
            ).wait()

        @pl.when(nraw <= CAP)
        def _():
            tail_cp.wait()

        tail_off = TAIL_START - lo
        tbase = base_s[62]
        tn = cnt_s[62]
        ngt = lax.shift_right_logical(tn + 15, 4)

        @pl.loop(0, ngt)
        def tail_grp(u):
            pv = sp_v[pl.ds(tbase + u * 16, 16)]
            for lane in range(16):
                jj = u * 16 + lane

                @pl.when(jj < tn)
                def _():
                    pk = pv[lane]
                    emit(jnp.bitwise_and(pk, 16383), tail_v,
                         lax.shift_right_logical(pk, 14) - tail_off)

        @pl.when(m_s[0] > 0)
        def _():
            flush()

    out = body(x.astype(jnp.int32), wt, tail)
    return lax.slice(out, (0, 0), (BATCH, DIM))
